# Initial kernel scaffold; baseline (speedup 1.0000x reference)
import jax
import jax.numpy as jnp
from jax.experimental import pallas as pl


def kernel(x, edge_index, W_in, b_in, W_mu0, b_mu0, W_mu, b_mu, W_si0, b_si0, W_si, b_si, W_zh, b_zh, W_rec0, b_rec0, W_out, b_out):
    raise NotImplementedError("write your pallas kernel here")


# speedup vs baseline: 5.4826x; 5.4826x over previous
import functools

import jax
import jax.numpy as jnp
from jax import lax
from jax.experimental import pallas as pl
from jax.experimental.pallas import tpu as pltpu
from jax.experimental.pallas import tpu_sc as plsc

N = 10000
E = 320000
IN_CH = 128
HID = 128
ZDIM = 32
NUM_CLASSES = 2

NUM_TILES = 16
NUM_CORES = 2
CHUNK = 128
CHUNKS_PER_TILE = 160
E_PAD = CHUNKS_PER_TILE * NUM_TILES * CHUNK
ROWS_PER_TILE = 632
N_PAD = N + 8
GROUP = 32
NUM_GROUPS = CHUNKS_PER_TILE // GROUP

ROW_BLK = 2000


def _sc_mesh():
    return plsc.VectorSubcoreMesh(core_axis_name="c", subcore_axis_name="s")


@functools.cache
def _make_agg(fc):
    @functools.partial(
        pl.kernel,
        mesh=_sc_mesh(),
        compiler_params=pltpu.CompilerParams(use_tc_tiling_on_sc=False),
        out_type=[jax.ShapeDtypeStruct((N, fc), jnp.float32),
                  jax.ShapeDtypeStruct((N, fc), jnp.float32)],
        scratch_types=[
            pltpu.VMEM_SHARED((N_PAD, fc), jnp.float32),
            pltpu.VMEM((GROUP, CHUNK), jnp.int32),
            pltpu.VMEM((GROUP, CHUNK), jnp.int32),
            pltpu.VMEM((CHUNK, fc), jnp.float32),
            pltpu.SemaphoreType.DMA,
        ],
    )
    def agg(y_l, y_r, src2d, dst2d, s_l, s_r, acc, srcv, dstv, rowbuf, sem):
        c = lax.axis_index("c")
        s = lax.axis_index("s")
        row0 = jnp.minimum(s * ROWS_PER_TILE, N - ROWS_PER_TILE)

        def run(y_hbm, out_hbm):
            pltpu.sync_copy(y_hbm.at[pl.ds(row0, ROWS_PER_TILE)],
                            acc.at[pl.ds(row0, ROWS_PER_TILE)])
            plsc.subcore_barrier()

            def group(g, carry):
                base = s * CHUNKS_PER_TILE + g * GROUP
                pltpu.sync_copy(src2d.at[pl.ds(base, GROUP)], srcv)
                pltpu.sync_copy(dst2d.at[pl.ds(base, GROUP)], dstv)

                def step(j, carry2):
                    pltpu.async_copy(y_hbm.at[srcv.at[j]], rowbuf, sem).wait()
                    pltpu.sync_copy(rowbuf, acc.at[dstv.at[j]], add=True)
                    return carry2

                return lax.fori_loop(0, GROUP, step, carry)

            lax.fori_loop(0, NUM_GROUPS, group, 0)
            plsc.subcore_barrier()
            pltpu.sync_copy(acc.at[pl.ds(row0, ROWS_PER_TILE)],
                            out_hbm.at[pl.ds(row0, ROWS_PER_TILE)])

        @pl.when(c == 0)
        def _():
            run(y_l, s_l)

        @pl.when(c == 1)
        def _():
            run(y_r, s_r)

    return agg


def _act(a, kind):
    if kind == "relu":
        return jnp.maximum(a, 0.0)
    if kind == "sigmoid":
        return jax.nn.sigmoid(a)
    return a


def _row_spec(width):
    return pl.BlockSpec((ROW_BLK, width), lambda i: (i, 0))


def _full_spec(shape):
    return pl.BlockSpec(shape, lambda i: (0, 0))


def _pre_body(x_ref, deg_ref, w_ref, yl_ref, yr_ref):
    dinv = lax.rsqrt(deg_ref[...][:, :1])
    y = jnp.dot(x_ref[...], w_ref[...], preferred_element_type=jnp.float32)
    y = y * dinv
    half = y.shape[1] // 2
    yl_ref[...] = y[:, :half]
    yr_ref[...] = y[:, half:]


def _pre(x, deg, w):
    f = w.shape[1]
    fc = f // 2
    return pl.pallas_call(
        _pre_body,
        grid=(N // ROW_BLK,),
        in_specs=[_row_spec(x.shape[1]), _row_spec(16), _full_spec(w.shape)],
        out_specs=[_row_spec(fc), _row_spec(fc)],
        out_shape=[jax.ShapeDtypeStruct((N, fc), jnp.float32)] * 2,
    )(x, deg, w)


def _comb_body(sl_ref, sr_ref, deg_ref, b_ref, w_ref, yl_ref, yr_ref, *, kind):
    dinv = lax.rsqrt(deg_ref[...][:, :1])
    sfull = jnp.concatenate([sl_ref[...], sr_ref[...]], axis=1)
    inp = _act(sfull * dinv + b_ref[...], kind)
    y = jnp.dot(inp, w_ref[...], preferred_element_type=jnp.float32)
    y = y * dinv
    half = y.shape[1] // 2
    yl_ref[...] = y[:, :half]
    yr_ref[...] = y[:, half:]


def _comb(sl, sr, deg, b, w, kind):
    fprev = 2 * sl.shape[1]
    f = w.shape[1]
    fc = f // 2
    return pl.pallas_call(
        functools.partial(_comb_body, kind=kind),
        grid=(N // ROW_BLK,),
        in_specs=[_row_spec(sl.shape[1]), _row_spec(sr.shape[1]),
                  _row_spec(16), _full_spec((1, fprev)), _full_spec(w.shape)],
        out_specs=[_row_spec(fc), _row_spec(fc)],
        out_shape=[jax.ShapeDtypeStruct((N, fc), jnp.float32)] * 2,
    )(sl, sr, deg, b.reshape(1, fprev), w)


def _act_out_body(sl_ref, sr_ref, deg_ref, b_ref, o_ref, *, kind):
    dinv = lax.rsqrt(deg_ref[...][:, :1])
    sfull = jnp.concatenate([sl_ref[...], sr_ref[...]], axis=1)
    o_ref[...] = _act(sfull * dinv + b_ref[...], kind)


def _act_out(sl, sr, deg, b, kind):
    f = 2 * sl.shape[1]
    return pl.pallas_call(
        functools.partial(_act_out_body, kind=kind),
        grid=(N // ROW_BLK,),
        in_specs=[_row_spec(sl.shape[1]), _row_spec(sr.shape[1]),
                  _row_spec(16), _full_spec((1, f))],
        out_specs=_row_spec(f),
        out_shape=jax.ShapeDtypeStruct((N, f), jnp.float32),
    )(sl, sr, deg, b.reshape(1, f))


def _zpre_body(mu_ref, si_ref, e_ref, deg_ref, w_ref, yl_ref, yr_ref):
    dinv = lax.rsqrt(deg_ref[...][:, :1])
    z = mu_ref[...] + si_ref[...] * e_ref[...]
    y = jnp.dot(z, w_ref[...], preferred_element_type=jnp.float32)
    y = y * dinv
    half = y.shape[1] // 2
    yl_ref[...] = y[:, :half]
    yr_ref[...] = y[:, half:]


def _zpre(mu, si, e, deg, w):
    f = w.shape[1]
    fc = f // 2
    return pl.pallas_call(
        _zpre_body,
        grid=(N // ROW_BLK,),
        in_specs=[_row_spec(ZDIM), _row_spec(ZDIM), _row_spec(ZDIM),
                  _row_spec(16), _full_spec(w.shape)],
        out_specs=[_row_spec(fc), _row_spec(fc)],
        out_shape=[jax.ShapeDtypeStruct((N, fc), jnp.float32)] * 2,
    )(mu, si, e, deg, w)


def kernel(x, edge_index, W_in, b_in, W_mu0, b_mu0, W_mu, b_mu, W_si0, b_si0,
           W_si, b_si, W_zh, b_zh, W_rec0, b_rec0, W_out, b_out):
    src = edge_index[0].astype(jnp.int32)
    dst = edge_index[1].astype(jnp.int32)
    pad = E_PAD - E
    src2d = jnp.concatenate(
        [src, jnp.zeros((pad,), jnp.int32)]).reshape(E_PAD // CHUNK, CHUNK)
    dst2d = jnp.concatenate(
        [dst, jnp.full((pad,), N, jnp.int32)]).reshape(E_PAD // CHUNK, CHUNK)
    agg64 = _make_agg(64)
    agg16 = _make_agg(16)
    agg128 = _make_agg(128)

    ones16 = jnp.ones((N, 16), jnp.float32)
    deg = agg16(ones16, ones16, src2d, dst2d)[0]

    y = _pre(x, deg, W_in)
    s_in = agg64(*y, src2d, dst2d)
    y = _comb(*s_in, deg, b_in, W_mu0, "relu")
    s_mu0 = agg64(*y, src2d, dst2d)
    y = _comb(*s_mu0, deg, b_mu0, W_mu, "relu")
    s_mu = agg16(*y, src2d, dst2d)
    mu = _act_out(*s_mu, deg, b_mu, "relu")
    y = _comb(*s_in, deg, b_in, W_si0, "relu")
    s_si0 = agg64(*y, src2d, dst2d)
    y = _comb(*s_si0, deg, b_si0, W_si, "relu")
    s_si = agg16(*y, src2d, dst2d)
    si = _act_out(*s_si, deg, b_si, "sigmoid")
    e = jax.random.normal(jax.random.key(1), (N, ZDIM), jnp.float32)
    y = _zpre(mu, si, e, deg, W_zh)
    s_zh = agg64(*y, src2d, dst2d)
    y = _comb(*s_zh, deg, b_zh, W_rec0, "none")
    s_rec = agg64(*y, src2d, dst2d)
    y = _comb(*s_rec, deg, b_rec0, W_out, "relu")
    s_out = agg128(*y, src2d, dst2d)
    logits = _act_out(*s_out, deg, b_out, "none")
    logits = logits.reshape(-1, NUM_CLASSES)
    return (logits, mu, si)


# speedup vs baseline: 5.9501x; 1.0853x over previous
import functools

import jax
import jax.numpy as jnp
from jax import lax
from jax.experimental import pallas as pl
from jax.experimental.pallas import tpu as pltpu
from jax.experimental.pallas import tpu_sc as plsc

N = 10000
E = 320000
IN_CH = 128
HID = 128
ZDIM = 32
NUM_CLASSES = 2

NUM_TILES = 16
CHUNK = 128
CHUNKS_PER_TILE = 160
E_PAD = CHUNKS_PER_TILE * NUM_TILES * CHUNK
ROWS_PER_TILE = 632
N_PAD = N + 8
KDEPTH = 4

ROW_BLK = 2000


def _sc_mesh():
    return plsc.VectorSubcoreMesh(core_axis_name="c", subcore_axis_name="s")


@functools.cache
def _make_agg(fc):
    @functools.partial(
        pl.kernel,
        mesh=_sc_mesh(),
        compiler_params=pltpu.CompilerParams(use_tc_tiling_on_sc=False),
        out_type=[jax.ShapeDtypeStruct((N, fc), jnp.float32),
                  jax.ShapeDtypeStruct((N, fc), jnp.float32)],
        scratch_types=[
            pltpu.VMEM_SHARED((N_PAD, fc), jnp.float32),
            pltpu.VMEM((CHUNKS_PER_TILE, CHUNK), jnp.int32),
            pltpu.VMEM((CHUNKS_PER_TILE, CHUNK), jnp.int32),
            pltpu.VMEM((KDEPTH, CHUNK, fc), jnp.float32),
            pltpu.SemaphoreType.DMA,
        ],
    )
    def agg(y_l, y_r, src2d, dst2d, s_l, s_r, acc, srcv, dstv, rb, sem):
        c = lax.axis_index("c")
        s = lax.axis_index("s")
        row0 = jnp.minimum(s * ROWS_PER_TILE, N - ROWS_PER_TILE)

        def run(y_hbm, out_hbm):
            pltpu.sync_copy(
                src2d.at[pl.ds(s * CHUNKS_PER_TILE, CHUNKS_PER_TILE)], srcv)
            pltpu.sync_copy(
                dst2d.at[pl.ds(s * CHUNKS_PER_TILE, CHUNKS_PER_TILE)], dstv)
            pltpu.sync_copy(y_hbm.at[pl.ds(row0, ROWS_PER_TILE)],
                            acc.at[pl.ds(row0, ROWS_PER_TILE)])
            plsc.subcore_barrier()

            def block(b, carry):
                base = b * KDEPTH
                descs = [
                    pltpu.async_copy(y_hbm.at[srcv.at[base + i]], rb.at[i],
                                     sem)
                    for i in range(KDEPTH)
                ]
                for d in descs:
                    d.wait()
                for i in range(KDEPTH):
                    pltpu.sync_copy(rb.at[i], acc.at[dstv.at[base + i]],
                                    add=True)
                return carry

            lax.fori_loop(0, CHUNKS_PER_TILE // KDEPTH, block, 0)
            plsc.subcore_barrier()
            pltpu.sync_copy(acc.at[pl.ds(row0, ROWS_PER_TILE)],
                            out_hbm.at[pl.ds(row0, ROWS_PER_TILE)])

        @pl.when(c == 0)
        def _():
            run(y_l, s_l)

        @pl.when(c == 1)
        def _():
            run(y_r, s_r)

    return agg


def _act(a, kind):
    if kind == "relu":
        return jnp.maximum(a, 0.0)
    if kind == "sigmoid":
        return jax.nn.sigmoid(a)
    return a


def _row_spec(width):
    return pl.BlockSpec((ROW_BLK, width), lambda i: (i, 0))


def _full_spec(shape):
    return pl.BlockSpec(shape, lambda i: (0, 0))


def _split_out(y, refs):
    fc = refs[0].shape[1]
    for k, ref in enumerate(refs):
        ref[...] = y[:, k * fc:(k + 1) * fc]


def _nparts(f):
    return 4 if f > HID else 2


def _pre_body(x_ref, deg_ref, w_ref, *y_refs):
    dinv = lax.rsqrt(deg_ref[...][:, :1])
    y = jnp.dot(x_ref[...], w_ref[...], preferred_element_type=jnp.float32)
    _split_out(y * dinv, y_refs)


def _pre(x, deg, w):
    f = w.shape[1]
    fc = f // _nparts(f)
    return pl.pallas_call(
        _pre_body,
        grid=(N // ROW_BLK,),
        in_specs=[_row_spec(x.shape[1]), _row_spec(16), _full_spec(w.shape)],
        out_specs=[_row_spec(fc)] * _nparts(f),
        out_shape=[jax.ShapeDtypeStruct((N, fc), jnp.float32)] * _nparts(f),
    )(x, deg, w)


def _comb_body(*refs, kind, nin):
    s_refs = refs[:nin]
    deg_ref, b_ref, w_ref = refs[nin:nin + 3]
    y_refs = refs[nin + 3:]
    dinv = lax.rsqrt(deg_ref[...][:, :1])
    sfull = jnp.concatenate([r[...] for r in s_refs], axis=1)
    inp = _act(sfull * dinv + b_ref[...], kind)
    y = jnp.dot(inp, w_ref[...], preferred_element_type=jnp.float32)
    _split_out(y * dinv, y_refs)


def _comb(s_parts, deg, b, w, kind):
    fprev = sum(p.shape[1] for p in s_parts)
    f = w.shape[1]
    fc = f // _nparts(f)
    return pl.pallas_call(
        functools.partial(_comb_body, kind=kind, nin=len(s_parts)),
        grid=(N // ROW_BLK,),
        in_specs=[_row_spec(p.shape[1]) for p in s_parts]
        + [_row_spec(16), _full_spec((1, fprev)), _full_spec(w.shape)],
        out_specs=[_row_spec(fc)] * _nparts(f),
        out_shape=[jax.ShapeDtypeStruct((N, fc), jnp.float32)] * _nparts(f),
    )(*s_parts, deg, b.reshape(1, fprev), w)


def _act_out_body(*refs, kind, nin):
    s_refs = refs[:nin]
    deg_ref, b_ref, o_ref = refs[nin:]
    dinv = lax.rsqrt(deg_ref[...][:, :1])
    sfull = jnp.concatenate([r[...] for r in s_refs], axis=1)
    o_ref[...] = _act(sfull * dinv + b_ref[...], kind)


def _act_out(s_parts, deg, b, kind):
    f = sum(p.shape[1] for p in s_parts)
    return pl.pallas_call(
        functools.partial(_act_out_body, kind=kind, nin=len(s_parts)),
        grid=(N // ROW_BLK,),
        in_specs=[_row_spec(p.shape[1]) for p in s_parts]
        + [_row_spec(16), _full_spec((1, f))],
        out_specs=_row_spec(f),
        out_shape=jax.ShapeDtypeStruct((N, f), jnp.float32),
    )(*s_parts, deg, b.reshape(1, f))


def _zpre_body(mu_ref, si_ref, e_ref, deg_ref, w_ref, *y_refs):
    dinv = lax.rsqrt(deg_ref[...][:, :1])
    z = mu_ref[...] + si_ref[...] * e_ref[...]
    y = jnp.dot(z, w_ref[...], preferred_element_type=jnp.float32)
    _split_out(y * dinv, y_refs)


def _zpre(mu, si, e, deg, w):
    f = w.shape[1]
    fc = f // _nparts(f)
    return pl.pallas_call(
        _zpre_body,
        grid=(N // ROW_BLK,),
        in_specs=[_row_spec(ZDIM), _row_spec(ZDIM), _row_spec(ZDIM),
                  _row_spec(16), _full_spec(w.shape)],
        out_specs=[_row_spec(fc)] * _nparts(f),
        out_shape=[jax.ShapeDtypeStruct((N, fc), jnp.float32)] * _nparts(f),
    )(mu, si, e, deg, w)


def kernel(x, edge_index, W_in, b_in, W_mu0, b_mu0, W_mu, b_mu, W_si0, b_si0,
           W_si, b_si, W_zh, b_zh, W_rec0, b_rec0, W_out, b_out):
    src = edge_index[0].astype(jnp.int32)
    dst = edge_index[1].astype(jnp.int32)
    pad = E_PAD - E
    src2d = jnp.concatenate(
        [src, jnp.zeros((pad,), jnp.int32)]).reshape(E_PAD // CHUNK, CHUNK)
    dst2d = jnp.concatenate(
        [dst, jnp.full((pad,), N, jnp.int32)]).reshape(E_PAD // CHUNK, CHUNK)

    agg64 = _make_agg(64)
    agg16 = _make_agg(16)

    ones16 = jnp.ones((N, 16), jnp.float32)
    deg = agg16(ones16, ones16, src2d, dst2d)[0]

    y = _pre(x, deg, W_in)
    s_in = agg64(*y, src2d, dst2d)
    y = _comb(s_in, deg, b_in, W_mu0, "relu")
    s_mu0 = agg64(*y, src2d, dst2d)
    y = _comb(s_mu0, deg, b_mu0, W_mu, "relu")
    s_mu = agg16(*y, src2d, dst2d)
    mu = _act_out(s_mu, deg, b_mu, "relu")
    y = _comb(s_in, deg, b_in, W_si0, "relu")
    s_si0 = agg64(*y, src2d, dst2d)
    y = _comb(s_si0, deg, b_si0, W_si, "relu")
    s_si = agg16(*y, src2d, dst2d)
    si = _act_out(s_si, deg, b_si, "sigmoid")
    e = jax.random.normal(jax.random.key(1), (N, ZDIM), jnp.float32)
    y = _zpre(mu, si, e, deg, W_zh)
    s_zh = agg64(*y, src2d, dst2d)
    y = _comb(s_zh, deg, b_zh, W_rec0, "none")
    s_rec = agg64(*y, src2d, dst2d)
    y = _comb(s_rec, deg, b_rec0, W_out, "relu")
    s_a = agg64(y[0], y[1], src2d, dst2d)
    s_b = agg64(y[2], y[3], src2d, dst2d)
    logits = _act_out([s_a[0], s_a[1], s_b[0], s_b[1]], deg, b_out, "none")
    logits = logits.reshape(-1, NUM_CLASSES)
    return (logits, mu, si)
